# baseline (speedup 1.0000x reference)
import jax
import jax.numpy as jnp
from jax import lax
from jax.experimental import pallas as pl
from jax.experimental.pallas import tpu as pltpu
from jax.experimental.pallas import tpu_sc as plsc

N = 10000
E = 320000
DIN = 128
H = 64
OUT = 24
SEQ = 72

NC, NS = 2, 16
NW = NC * NS
EPS = E // NW
CH = 80
NCH = EPS // CH
RING = 5
SLC = 640
SLL = N - 15 * SLC
ZR = 80

RB = 8000
SCR = 320000
TCR = N * H - SCR
NB = TCR // RB
PW = SCR // NW
CW = 400
NCW = PW // CW
BLK = 10000

_mesh = plsc.VectorSubcoreMesh(core_axis_name="c", subcore_axis_name="s")


def _deg_body(eir_hbm, out_hbm, idxd_v, ones_v, zb_v, acc_sh, sem):
    c = lax.axis_index("c")
    s = lax.axis_index("s")
    w = c * NS + s
    pltpu.sync_copy(eir_hbm.at[1, w], idxd_v)

    one16 = jnp.full((16,), 1.0, jnp.float32)
    z16 = jnp.zeros((16,), jnp.float32)

    def fill_ones(i, carry):
        ones_v[i, :] = one16
        return carry
    lax.fori_loop(0, CH, fill_ones, 0)

    def fill_z(i, carry):
        zb_v[i, :] = z16
        return carry
    lax.fori_loop(0, ZR, fill_z, 0)
    for q in range(SLC // ZR):
        if q < SLL // ZR:
            pltpu.sync_copy(zb_v, acc_sh.at[pl.ds(s * SLC + q * ZR, ZR)])
        else:
            @pl.when(s < NS - 1)
            def _():
                pltpu.sync_copy(zb_v, acc_sh.at[pl.ds(s * SLC + q * ZR, ZR)])
    plsc.subcore_barrier()

    def fire(j, carry):
        pltpu.async_copy(ones_v, acc_sh.at[idxd_v.at[j]], sem, add=True)
        return carry
    lax.fori_loop(0, NCH, fire, 0)

    def drain(j, carry):
        pltpu.make_async_copy(ones_v, acc_sh.at[idxd_v.at[0]], sem).wait()
        return carry
    lax.fori_loop(0, NCH, drain, 0)
    plsc.subcore_barrier()

    @pl.when(s < NS - 1)
    def _():
        pltpu.sync_copy(acc_sh.at[pl.ds(s * SLC, SLC)],
                        out_hbm.at[pl.ds(c * N + s * SLC, SLC)])

    @pl.when(s == NS - 1)
    def _():
        pltpu.sync_copy(acc_sh.at[pl.ds((NS - 1) * SLC, SLL)],
                        out_hbm.at[pl.ds(c * N + (NS - 1) * SLC, SLL)])


def _sc_degree(eir):
    return pl.kernel(
        _deg_body,
        out_type=jax.ShapeDtypeStruct((NC * N, 16), jnp.float32),
        mesh=_mesh,
        compiler_params=pltpu.CompilerParams(use_tc_tiling_on_sc=False),
        scratch_types=[
            pltpu.VMEM((NCH, CH), jnp.int32),
            pltpu.VMEM((CH, 16), jnp.float32),
            pltpu.VMEM((ZR, 16), jnp.float32),
            pltpu.VMEM_SHARED((N, 16), jnp.float32),
            pltpu.SemaphoreType.DMA,
        ],
    )(eir)


def _seg_body(eir_hbm, g_hbm, out_hbm,
              idxs_v, idxd_v, rows_v, zb_v, acc_sh, gsem, ssem):
    c = lax.axis_index("c")
    s = lax.axis_index("s")
    w = c * NS + s
    pltpu.sync_copy(eir_hbm.at[0, w], idxs_v)
    pltpu.sync_copy(eir_hbm.at[1, w], idxd_v)

    z16 = jnp.zeros((16,), jnp.float32)

    def fill_z(i, carry):
        for k in range(H // 16):
            zb_v[i, pl.ds(k * 16, 16)] = z16
        return carry
    lax.fori_loop(0, ZR, fill_z, 0)
    for q in range(SLC // ZR):
        if q < SLL // ZR:
            pltpu.sync_copy(zb_v, acc_sh.at[pl.ds(s * SLC + q * ZR, ZR)])
        else:
            @pl.when(s < NS - 1)
            def _():
                pltpu.sync_copy(zb_v, acc_sh.at[pl.ds(s * SLC + q * ZR, ZR)])
    plsc.subcore_barrier()

    def start_gather(j, p):
        pltpu.async_copy(g_hbm.at[idxs_v.at[j]], rows_v.at[p], gsem.at[p])

    def wait_gather(p):
        pltpu.make_async_copy(g_hbm.at[idxs_v.at[0]], rows_v.at[p],
                              gsem.at[p]).wait()

    def start_scatter(j, p):
        pltpu.async_copy(rows_v.at[p], acc_sh.at[idxd_v.at[j]], ssem.at[p],
                         add=True)

    def wait_scatter(p):
        pltpu.make_async_copy(rows_v.at[p], acc_sh.at[idxd_v.at[0]],
                              ssem.at[p]).wait()

    for p in range(RING - 1):
        start_gather(p, p)

    def outer(j5, carry):
        for p in range(RING):
            j = j5 * RING + p
            wait_gather(p)
            start_scatter(j, p)
            pn = (p + RING - 1) % RING
            jn = j + RING - 1

            @pl.when(jn < NCH)
            def _():
                @pl.when(j >= 1)
                def _():
                    wait_scatter(pn)
                start_gather(jn, pn)
        return carry
    lax.fori_loop(0, NCH // RING, outer, 0)

    for p in range(RING):
        wait_scatter(p)
    plsc.subcore_barrier()

    @pl.when(s < NS - 1)
    def _():
        pltpu.sync_copy(acc_sh.at[pl.ds(s * SLC, SLC)],
                        out_hbm.at[pl.ds(c * N + s * SLC, SLC)])

    @pl.when(s == NS - 1)
    def _():
        pltpu.sync_copy(acc_sh.at[pl.ds((NS - 1) * SLC, SLL)],
                        out_hbm.at[pl.ds(c * N + (NS - 1) * SLC, SLL)])


def _sc_segsum(eir, g):
    return pl.kernel(
        _seg_body,
        out_type=jax.ShapeDtypeStruct((NC * N, H), jnp.float32),
        mesh=_mesh,
        compiler_params=pltpu.CompilerParams(use_tc_tiling_on_sc=False),
        scratch_types=[
            pltpu.VMEM((NCH, CH), jnp.int32),
            pltpu.VMEM((NCH, CH), jnp.int32),
            pltpu.VMEM((RING, CH, H), jnp.float32),
            pltpu.VMEM((ZR, H), jnp.float32),
            pltpu.VMEM_SHARED((N, H), jnp.float32),
            pltpu.SemaphoreType.DMA((RING,)),
            pltpu.SemaphoreType.DMA((RING,)),
        ],
    )(eir, g)


def _mv_body(wc_hbm, c_hbm, out_hbm, cbuf_v, wbuf_v, accb_v, sem):
    cx = lax.axis_index("c")
    s = lax.axis_index("s")
    w = cx * NS + s
    r0 = TCR + w * PW
    pltpu.sync_copy(c_hbm.at[pl.ds(r0, PW)], cbuf_v)

    def start_chunk(j, b):
        pltpu.async_copy(wc_hbm.at[pl.ds(r0 + j * CW, CW)], wbuf_v.at[b],
                         sem.at[b])

    def wait_chunk(b):
        pltpu.make_async_copy(wc_hbm.at[pl.ds(r0, CW)], wbuf_v.at[b],
                              sem.at[b]).wait()

    start_chunk(0, 0)
    z16 = jnp.zeros((16,), jnp.float32)
    accs0 = (z16,) * 8

    def outer(j2, accs):
        for b in range(2):
            j = j2 * 2 + b

            @pl.when(j + 1 < NCW)
            def _():
                start_chunk(j + 1, 1 - b)
            wait_chunk(b)

            def row_step(t, a):
                cv = cbuf_v[pl.ds(j * CW + t * 16, 16)]
                for m in range(16):
                    cb = jnp.full((16,), cv[m], jnp.float32)
                    a = tuple(
                        a[k] + cb * wbuf_v[b, t * 16 + m, pl.ds(16 * k, 16)]
                        for k in range(8))
                return a
            accs = lax.fori_loop(0, CW // 16, row_step, accs)
        return accs
    accs = lax.fori_loop(0, NCW // 2, outer, accs0)

    wait_chunk(0)

    def row_last(t, a):
        cv = cbuf_v[pl.ds((NCW - 1) * CW + t * 16, 16)]
        for m in range(16):
            cb = jnp.full((16,), cv[m], jnp.float32)
            a = tuple(
                a[k] + cb * wbuf_v[0, t * 16 + m, pl.ds(16 * k, 16)]
                for k in range(8))
        return a
    accs = lax.fori_loop(0, CW // 16, row_last, accs)

    for k in range(8):
        accb_v[0, pl.ds(16 * k, 16)] = accs[k]
    pltpu.sync_copy(accb_v, out_hbm.at[w])


def _sc_matvec(Wc, cflat):
    return pl.kernel(
        _mv_body,
        out_type=jax.ShapeDtypeStruct((NW, 1, 2 * H), jnp.float32),
        mesh=_mesh,
        scratch_types=[
            pltpu.VMEM((PW,), jnp.float32),
            pltpu.VMEM((2, CW, 2 * H), jnp.float32),
            pltpu.VMEM((1, 2 * H), jnp.float32),
            pltpu.SemaphoreType.DMA((2,)),
        ],
    )(Wc, cflat)


def _prep_body(degp_ref, x_ref, w1_ref, g1_ref, dis_ref):
    deg = 1.0 + degp_ref[0, :, 0:1] + degp_ref[1, :, 0:1]
    dis = lax.rsqrt(deg)
    dis_ref[...] = dis
    g1_ref[...] = jnp.dot(x_ref[...], w1_ref[...],
                          preferred_element_type=jnp.float32) * dis


def _tc_prep(degp, x, W1):
    return pl.pallas_call(
        _prep_body,
        grid=(N // BLK,),
        in_specs=[
            pl.BlockSpec((NC, BLK, 16), lambda i: (0, i, 0)),
            pl.BlockSpec((BLK, DIN), lambda i: (i, 0)),
            pl.BlockSpec((DIN, H), lambda i: (0, 0)),
        ],
        out_specs=[
            pl.BlockSpec((BLK, H), lambda i: (i, 0)),
            pl.BlockSpec((BLK, 1), lambda i: (i, 0)),
        ],
        out_shape=[
            jax.ShapeDtypeStruct((N, H), jnp.float32),
            jax.ShapeDtypeStruct((N, 1), jnp.float32),
        ],
    )(degp, x, W1)


def _mid_body(s1_ref, g1_ref, dis_ref, w2_ref, b1_ref, g2_ref):
    agg = s1_ref[0] + s1_ref[1] + g1_ref[...]
    h1 = jnp.maximum(agg * dis_ref[...] + b1_ref[...], 0.0)
    g2_ref[...] = jnp.dot(h1, w2_ref[...],
                          preferred_element_type=jnp.float32) * dis_ref[...]


def _tc_mid(s1, g1, dis, W2, b1):
    return pl.pallas_call(
        _mid_body,
        grid=(N // BLK,),
        in_specs=[
            pl.BlockSpec((NC, BLK, H), lambda i: (0, i, 0)),
            pl.BlockSpec((BLK, H), lambda i: (i, 0)),
            pl.BlockSpec((BLK, 1), lambda i: (i, 0)),
            pl.BlockSpec((H, H), lambda i: (0, 0)),
            pl.BlockSpec((1, H), lambda i: (0, 0)),
        ],
        out_specs=pl.BlockSpec((BLK, H), lambda i: (i, 0)),
        out_shape=jax.ShapeDtypeStruct((N, H), jnp.float32),
    )(s1, g1, dis, W2, b1)


def _post_body(s2_ref, g2_ref, dis_ref, b2_ref, h2_ref):
    agg = s2_ref[0] + s2_ref[1] + g2_ref[...]
    h2 = jnp.maximum(agg * dis_ref[...] + b2_ref[...], 0.0)
    h2_ref[...] = jnp.minimum(h2, 10.0)


def _tc_post(s2, g2, dis, b2):
    return pl.pallas_call(
        _post_body,
        grid=(N // BLK,),
        in_specs=[
            pl.BlockSpec((NC, BLK, H), lambda i: (0, i, 0)),
            pl.BlockSpec((BLK, H), lambda i: (i, 0)),
            pl.BlockSpec((BLK, 1), lambda i: (i, 0)),
            pl.BlockSpec((1, H), lambda i: (0, 0)),
        ],
        out_specs=pl.BlockSpec((BLK, H), lambda i: (i, 0)),
        out_shape=jax.ShapeDtypeStruct((N, H), jnp.float32),
    )(s2, g2, dis, b2)


def _final_body(h2r_ref, wc_ref, out_ref, acc_ref):
    i = pl.program_id(0)

    @pl.when(i == 0)
    def _():
        acc_ref[...] = jnp.zeros_like(acc_ref)

    acc_ref[...] += jnp.dot(h2r_ref[0], wc_ref[...],
                            preferred_element_type=jnp.float32)

    @pl.when(i == NB - 1)
    def _():
        out_ref[...] = acc_ref[...]


def _mlp_head(h2r, Wc):
    return pl.pallas_call(
        _final_body,
        grid=(NB,),
        in_specs=[
            pl.BlockSpec((1, 1, RB), lambda i: (i, 0, 0)),
            pl.BlockSpec((RB, 2 * H), lambda i: (i, 0)),
        ],
        out_specs=pl.BlockSpec((1, 2 * H), lambda i: (0, 0)),
        out_shape=jax.ShapeDtypeStruct((1, 2 * H), jnp.float32),
        scratch_shapes=[pltpu.VMEM((1, 2 * H), jnp.float32)],
    )(h2r, Wc)


def _combine_body(tcp_ref, scp_ref, wtail_ref, weather_ref, wa_ref, ba_ref,
                  wb_ref, bb_ref, bc_ref, wd_ref, bd_ref, out_ref):
    wf = jnp.maximum(
        jnp.dot(weather_ref[...], wa_ref[...],
                preferred_element_type=jnp.float32) + ba_ref[...], 0.0)
    wf = jnp.dot(wf, wb_ref[...],
                 preferred_element_type=jnp.float32) + bb_ref[...]
    t = (tcp_ref[...] + jnp.sum(scp_ref[...], axis=0, keepdims=True)
         + jnp.dot(wf, wtail_ref[...], preferred_element_type=jnp.float32)
         + bc_ref[...])
    t = jnp.maximum(t, 0.0)
    out_ref[...] = jnp.dot(t, wd_ref[...],
                           preferred_element_type=jnp.float32) + bd_ref[...]


def _combine(tcp, scp, Wc, weather, Wa, ba, Wb, bb, bc, Wd, bd):
    full = lambda s: pl.BlockSpec(s, lambda i: (0, 0))
    return pl.pallas_call(
        _combine_body,
        grid=(1,),
        in_specs=[
            full((1, 2 * H)),
            full((NW, 2 * H)),
            pl.BlockSpec((H, 2 * H), lambda i: ((N * H) // H, 0)),
            full((1, SEQ + 1)),
            full((SEQ + 1, H)),
            full((1, H)),
            full((H, H)),
            full((1, H)),
            full((1, 2 * H)),
            full((2 * H, OUT)),
            full((1, OUT)),
        ],
        out_specs=pl.BlockSpec((1, OUT), lambda i: (0, 0)),
        out_shape=jax.ShapeDtypeStruct((1, OUT), jnp.float32),
    )(tcp, scp, Wc, weather, Wa, ba, Wb, bb, bc, Wd, bd)


def kernel(x, edge_index, rain_history, future_rain, W1, b1, W2, b2,
           Wa, ba, Wb, bb, Wc, bc, Wd, bd):
    eir = edge_index.astype(jnp.int32).reshape(2, NW, NCH, CH)

    degp = _sc_degree(eir).reshape(NC, N, 16)
    g1, dis = _tc_prep(degp, x, W1)
    s1 = _sc_segsum(eir, g1).reshape(NC, N, H)
    g2 = _tc_mid(s1, g1, dis, W2, b1.reshape(1, H))
    s2 = _sc_segsum(eir, g2).reshape(NC, N, H)
    h2 = _tc_post(s2, g2, dis, b2.reshape(1, H))

    cflat = h2.reshape(N * H)
    h2r = h2.reshape((N * H) // RB, 1, RB)
    scp = _sc_matvec(Wc, cflat).reshape(NW, 2 * H)
    tcp = _mlp_head(h2r, Wc)
    weather = jnp.concatenate([rain_history, future_rain], axis=1)
    out = _combine(tcp, scp, Wc, weather, Wa, ba.reshape(1, H),
                   Wb, bb.reshape(1, H), bc.reshape(1, 2 * H),
                   Wd, bd.reshape(1, OUT))
    return out[0]


# speedup vs baseline: 1.0072x; 1.0072x over previous
import jax
import jax.numpy as jnp
from jax import lax
from jax.experimental import pallas as pl
from jax.experimental.pallas import tpu as pltpu
from jax.experimental.pallas import tpu_sc as plsc

N = 10000
E = 320000
DIN = 128
H = 64
OUT = 24
SEQ = 72

NC, NS = 2, 16
NW = NC * NS
EPS = E // NW
CH = 80
NCH = EPS // CH
RING = 5
SLC = 640
SLL = N - 15 * SLC
ZR = 80

RB = 8000
SCR = 320000
TCR = N * H - SCR
NB = TCR // RB
PW = SCR // NW
CW = 400
NCW = PW // CW
BLK = 5000

_mesh = plsc.VectorSubcoreMesh(core_axis_name="c", subcore_axis_name="s")


def _deg_body(eir_hbm, out_hbm, idxd_v, ones_v, zb_v, acc_sh, sem):
    c = lax.axis_index("c")
    s = lax.axis_index("s")
    w = c * NS + s
    pltpu.sync_copy(eir_hbm.at[1, w], idxd_v)

    one16 = jnp.full((16,), 1.0, jnp.float32)
    z16 = jnp.zeros((16,), jnp.float32)

    def fill_ones(i, carry):
        ones_v[i, :] = one16
        return carry
    lax.fori_loop(0, CH, fill_ones, 0)

    def fill_z(i, carry):
        zb_v[i, :] = z16
        return carry
    lax.fori_loop(0, ZR, fill_z, 0)
    for q in range(SLC // ZR):
        if q < SLL // ZR:
            pltpu.sync_copy(zb_v, acc_sh.at[pl.ds(s * SLC + q * ZR, ZR)])
        else:
            @pl.when(s < NS - 1)
            def _():
                pltpu.sync_copy(zb_v, acc_sh.at[pl.ds(s * SLC + q * ZR, ZR)])
    plsc.subcore_barrier()

    def fire(j, carry):
        pltpu.async_copy(ones_v, acc_sh.at[idxd_v.at[j]], sem, add=True)
        return carry
    lax.fori_loop(0, NCH, fire, 0)

    def drain(j, carry):
        pltpu.make_async_copy(ones_v, acc_sh.at[idxd_v.at[0]], sem).wait()
        return carry
    lax.fori_loop(0, NCH, drain, 0)
    plsc.subcore_barrier()

    @pl.when(s < NS - 1)
    def _():
        pltpu.sync_copy(acc_sh.at[pl.ds(s * SLC, SLC)],
                        out_hbm.at[pl.ds(c * N + s * SLC, SLC)])

    @pl.when(s == NS - 1)
    def _():
        pltpu.sync_copy(acc_sh.at[pl.ds((NS - 1) * SLC, SLL)],
                        out_hbm.at[pl.ds(c * N + (NS - 1) * SLC, SLL)])


def _sc_degree(eir):
    return pl.kernel(
        _deg_body,
        out_type=jax.ShapeDtypeStruct((NC * N, 16), jnp.float32),
        mesh=_mesh,
        compiler_params=pltpu.CompilerParams(use_tc_tiling_on_sc=False),
        scratch_types=[
            pltpu.VMEM((NCH, CH), jnp.int32),
            pltpu.VMEM((CH, 16), jnp.float32),
            pltpu.VMEM((ZR, 16), jnp.float32),
            pltpu.VMEM_SHARED((N, 16), jnp.float32),
            pltpu.SemaphoreType.DMA,
        ],
    )(eir)


def _seg_body(eir_hbm, g_hbm, out_hbm,
              idxs_v, idxd_v, rows_v, zb_v, acc_sh, gsem, ssem):
    c = lax.axis_index("c")
    s = lax.axis_index("s")
    w = c * NS + s
    pltpu.sync_copy(eir_hbm.at[0, w], idxs_v)
    pltpu.sync_copy(eir_hbm.at[1, w], idxd_v)

    z16 = jnp.zeros((16,), jnp.float32)

    def fill_z(i, carry):
        for k in range(H // 16):
            zb_v[i, pl.ds(k * 16, 16)] = z16
        return carry
    lax.fori_loop(0, ZR, fill_z, 0)
    for q in range(SLC // ZR):
        if q < SLL // ZR:
            pltpu.sync_copy(zb_v, acc_sh.at[pl.ds(s * SLC + q * ZR, ZR)])
        else:
            @pl.when(s < NS - 1)
            def _():
                pltpu.sync_copy(zb_v, acc_sh.at[pl.ds(s * SLC + q * ZR, ZR)])
    plsc.subcore_barrier()

    def start_gather(j, p):
        pltpu.async_copy(g_hbm.at[idxs_v.at[j]], rows_v.at[p], gsem.at[p])

    def wait_gather(p):
        pltpu.make_async_copy(g_hbm.at[idxs_v.at[0]], rows_v.at[p],
                              gsem.at[p]).wait()

    def start_scatter(j, p):
        pltpu.async_copy(rows_v.at[p], acc_sh.at[idxd_v.at[j]], ssem.at[p],
                         add=True)

    def wait_scatter(p):
        pltpu.make_async_copy(rows_v.at[p], acc_sh.at[idxd_v.at[0]],
                              ssem.at[p]).wait()

    for p in range(RING - 1):
        start_gather(p, p)

    def outer(j5, carry):
        for p in range(RING):
            j = j5 * RING + p
            wait_gather(p)
            start_scatter(j, p)
            pn = (p + RING - 1) % RING
            jn = j + RING - 1

            @pl.when(jn < NCH)
            def _():
                @pl.when(j >= 1)
                def _():
                    wait_scatter(pn)
                start_gather(jn, pn)
        return carry
    lax.fori_loop(0, NCH // RING, outer, 0)

    for p in range(RING):
        wait_scatter(p)
    plsc.subcore_barrier()

    @pl.when(s < NS - 1)
    def _():
        pltpu.sync_copy(acc_sh.at[pl.ds(s * SLC, SLC)],
                        out_hbm.at[pl.ds(c * N + s * SLC, SLC)])

    @pl.when(s == NS - 1)
    def _():
        pltpu.sync_copy(acc_sh.at[pl.ds((NS - 1) * SLC, SLL)],
                        out_hbm.at[pl.ds(c * N + (NS - 1) * SLC, SLL)])


def _sc_segsum(eir, g):
    return pl.kernel(
        _seg_body,
        out_type=jax.ShapeDtypeStruct((NC * N, H), jnp.float32),
        mesh=_mesh,
        compiler_params=pltpu.CompilerParams(use_tc_tiling_on_sc=False),
        scratch_types=[
            pltpu.VMEM((NCH, CH), jnp.int32),
            pltpu.VMEM((NCH, CH), jnp.int32),
            pltpu.VMEM((RING, CH, H), jnp.float32),
            pltpu.VMEM((ZR, H), jnp.float32),
            pltpu.VMEM_SHARED((N, H), jnp.float32),
            pltpu.SemaphoreType.DMA((RING,)),
            pltpu.SemaphoreType.DMA((RING,)),
        ],
    )(eir, g)


def _mv_body(wc_hbm, c_hbm, out_hbm, cbuf_v, wbuf_v, accb_v, sem):
    cx = lax.axis_index("c")
    s = lax.axis_index("s")
    w = cx * NS + s
    r0 = TCR + w * PW
    pltpu.sync_copy(c_hbm.at[pl.ds(r0, PW)], cbuf_v)

    def start_chunk(j, b):
        pltpu.async_copy(wc_hbm.at[pl.ds(r0 + j * CW, CW)], wbuf_v.at[b],
                         sem.at[b])

    def wait_chunk(b):
        pltpu.make_async_copy(wc_hbm.at[pl.ds(r0, CW)], wbuf_v.at[b],
                              sem.at[b]).wait()

    start_chunk(0, 0)
    z16 = jnp.zeros((16,), jnp.float32)
    accs0 = (z16,) * 8

    def outer(j2, accs):
        for b in range(2):
            j = j2 * 2 + b

            @pl.when(j + 1 < NCW)
            def _():
                start_chunk(j + 1, 1 - b)
            wait_chunk(b)

            def row_step(t, a):
                cv = cbuf_v[pl.ds(j * CW + t * 16, 16)]
                for m in range(16):
                    cb = jnp.full((16,), cv[m], jnp.float32)
                    a = tuple(
                        a[k] + cb * wbuf_v[b, t * 16 + m, pl.ds(16 * k, 16)]
                        for k in range(8))
                return a
            accs = lax.fori_loop(0, CW // 16, row_step, accs)
        return accs
    accs = lax.fori_loop(0, NCW // 2, outer, accs0)

    wait_chunk(0)

    def row_last(t, a):
        cv = cbuf_v[pl.ds((NCW - 1) * CW + t * 16, 16)]
        for m in range(16):
            cb = jnp.full((16,), cv[m], jnp.float32)
            a = tuple(
                a[k] + cb * wbuf_v[0, t * 16 + m, pl.ds(16 * k, 16)]
                for k in range(8))
        return a
    accs = lax.fori_loop(0, CW // 16, row_last, accs)

    for k in range(8):
        accb_v[0, pl.ds(16 * k, 16)] = accs[k]
    pltpu.sync_copy(accb_v, out_hbm.at[w])


def _sc_matvec(Wc, cflat):
    return pl.kernel(
        _mv_body,
        out_type=jax.ShapeDtypeStruct((NW, 1, 2 * H), jnp.float32),
        mesh=_mesh,
        scratch_types=[
            pltpu.VMEM((PW,), jnp.float32),
            pltpu.VMEM((2, CW, 2 * H), jnp.float32),
            pltpu.VMEM((1, 2 * H), jnp.float32),
            pltpu.SemaphoreType.DMA((2,)),
        ],
    )(Wc, cflat)


def _prep_body(degp_ref, x_ref, w1_ref, g1_ref, dis_ref):
    deg = 1.0 + degp_ref[0, :, 0:1] + degp_ref[1, :, 0:1]
    dis = lax.rsqrt(deg)
    dis_ref[...] = dis
    g1_ref[...] = jnp.dot(x_ref[...], w1_ref[...],
                          preferred_element_type=jnp.float32) * dis


def _tc_prep(degp, x, W1):
    return pl.pallas_call(
        _prep_body,
        grid=(N // BLK,),
        in_specs=[
            pl.BlockSpec((NC, BLK, 16), lambda i: (0, i, 0)),
            pl.BlockSpec((BLK, DIN), lambda i: (i, 0)),
            pl.BlockSpec((DIN, H), lambda i: (0, 0)),
        ],
        out_specs=[
            pl.BlockSpec((BLK, H), lambda i: (i, 0)),
            pl.BlockSpec((BLK, 1), lambda i: (i, 0)),
        ],
        out_shape=[
            jax.ShapeDtypeStruct((N, H), jnp.float32),
            jax.ShapeDtypeStruct((N, 1), jnp.float32),
        ],
    )(degp, x, W1)


def _mid_body(s1_ref, g1_ref, dis_ref, w2_ref, b1_ref, g2_ref):
    agg = s1_ref[0] + s1_ref[1] + g1_ref[...]
    h1 = jnp.maximum(agg * dis_ref[...] + b1_ref[...], 0.0)
    g2_ref[...] = jnp.dot(h1, w2_ref[...],
                          preferred_element_type=jnp.float32) * dis_ref[...]


def _tc_mid(s1, g1, dis, W2, b1):
    return pl.pallas_call(
        _mid_body,
        grid=(N // BLK,),
        in_specs=[
            pl.BlockSpec((NC, BLK, H), lambda i: (0, i, 0)),
            pl.BlockSpec((BLK, H), lambda i: (i, 0)),
            pl.BlockSpec((BLK, 1), lambda i: (i, 0)),
            pl.BlockSpec((H, H), lambda i: (0, 0)),
            pl.BlockSpec((1, H), lambda i: (0, 0)),
        ],
        out_specs=pl.BlockSpec((BLK, H), lambda i: (i, 0)),
        out_shape=jax.ShapeDtypeStruct((N, H), jnp.float32),
    )(s1, g1, dis, W2, b1)


def _post_body(s2_ref, g2_ref, dis_ref, b2_ref, h2_ref):
    agg = s2_ref[0] + s2_ref[1] + g2_ref[...]
    h2 = jnp.maximum(agg * dis_ref[...] + b2_ref[...], 0.0)
    h2_ref[...] = jnp.minimum(h2, 10.0)


def _tc_post(s2, g2, dis, b2):
    return pl.pallas_call(
        _post_body,
        grid=(N // BLK,),
        in_specs=[
            pl.BlockSpec((NC, BLK, H), lambda i: (0, i, 0)),
            pl.BlockSpec((BLK, H), lambda i: (i, 0)),
            pl.BlockSpec((BLK, 1), lambda i: (i, 0)),
            pl.BlockSpec((1, H), lambda i: (0, 0)),
        ],
        out_specs=pl.BlockSpec((BLK, H), lambda i: (i, 0)),
        out_shape=jax.ShapeDtypeStruct((N, H), jnp.float32),
    )(s2, g2, dis, b2)


def _final_body(h2r_ref, wc_ref, out_ref, acc_ref):
    i = pl.program_id(0)

    @pl.when(i == 0)
    def _():
        acc_ref[...] = jnp.zeros_like(acc_ref)

    acc_ref[...] += jnp.dot(h2r_ref[0], wc_ref[...],
                            preferred_element_type=jnp.float32)

    @pl.when(i == NB - 1)
    def _():
        out_ref[...] = acc_ref[...]


def _mlp_head(h2r, Wc):
    return pl.pallas_call(
        _final_body,
        grid=(NB,),
        in_specs=[
            pl.BlockSpec((1, 1, RB), lambda i: (i, 0, 0)),
            pl.BlockSpec((RB, 2 * H), lambda i: (i, 0)),
        ],
        out_specs=pl.BlockSpec((1, 2 * H), lambda i: (0, 0)),
        out_shape=jax.ShapeDtypeStruct((1, 2 * H), jnp.float32),
        scratch_shapes=[pltpu.VMEM((1, 2 * H), jnp.float32)],
    )(h2r, Wc)


def _combine_body(tcp_ref, scp_ref, wtail_ref, weather_ref, wa_ref, ba_ref,
                  wb_ref, bb_ref, bc_ref, wd_ref, bd_ref, out_ref):
    wf = jnp.maximum(
        jnp.dot(weather_ref[...], wa_ref[...],
                preferred_element_type=jnp.float32) + ba_ref[...], 0.0)
    wf = jnp.dot(wf, wb_ref[...],
                 preferred_element_type=jnp.float32) + bb_ref[...]
    t = (tcp_ref[...] + jnp.sum(scp_ref[...], axis=0, keepdims=True)
         + jnp.dot(wf, wtail_ref[...], preferred_element_type=jnp.float32)
         + bc_ref[...])
    t = jnp.maximum(t, 0.0)
    out_ref[...] = jnp.dot(t, wd_ref[...],
                           preferred_element_type=jnp.float32) + bd_ref[...]


def _combine(tcp, scp, Wc, weather, Wa, ba, Wb, bb, bc, Wd, bd):
    full = lambda s: pl.BlockSpec(s, lambda i: (0, 0))
    return pl.pallas_call(
        _combine_body,
        grid=(1,),
        in_specs=[
            full((1, 2 * H)),
            full((NW, 2 * H)),
            pl.BlockSpec((H, 2 * H), lambda i: ((N * H) // H, 0)),
            full((1, SEQ + 1)),
            full((SEQ + 1, H)),
            full((1, H)),
            full((H, H)),
            full((1, H)),
            full((1, 2 * H)),
            full((2 * H, OUT)),
            full((1, OUT)),
        ],
        out_specs=pl.BlockSpec((1, OUT), lambda i: (0, 0)),
        out_shape=jax.ShapeDtypeStruct((1, OUT), jnp.float32),
    )(tcp, scp, Wc, weather, Wa, ba, Wb, bb, bc, Wd, bd)


def kernel(x, edge_index, rain_history, future_rain, W1, b1, W2, b2,
           Wa, ba, Wb, bb, Wc, bc, Wd, bd):
    eir = edge_index.astype(jnp.int32).reshape(2, NW, NCH, CH)

    degp = _sc_degree(eir).reshape(NC, N, 16)
    g1, dis = _tc_prep(degp, x, W1)
    s1 = _sc_segsum(eir, g1).reshape(NC, N, H)
    g2 = _tc_mid(s1, g1, dis, W2, b1.reshape(1, H))
    s2 = _sc_segsum(eir, g2).reshape(NC, N, H)
    h2 = _tc_post(s2, g2, dis, b2.reshape(1, H))

    cflat = h2.reshape(N * H)
    h2r = h2.reshape((N * H) // RB, 1, RB)
    scp = _sc_matvec(Wc, cflat).reshape(NW, 2 * H)
    tcp = _mlp_head(h2r, Wc)
    weather = jnp.concatenate([rain_history, future_rain], axis=1)
    out = _combine(tcp, scp, Wc, weather, Wa, ba.reshape(1, H),
                   Wb, bb.reshape(1, H), bc.reshape(1, 2 * H),
                   Wd, bd.reshape(1, OUT))
    return out[0]
